# Initial kernel scaffold; baseline (speedup 1.0000x reference)
import jax
import jax.numpy as jnp
from jax.experimental import pallas as pl


def kernel(pos_batch, neg_batch, entity_emb, relation_emb):
    raise NotImplementedError("write your pallas kernel here")


# speedup vs baseline: 1.1943x; 1.1943x over previous
import functools

import jax
import jax.numpy as jnp
from jax import lax
from jax.experimental import pallas as pl
from jax.experimental.pallas import tpu as pltpu
from jax.experimental.pallas import tpu_sc as plsc

_B = 16384
_D = 64
_MARGIN = 1.0
_NC = 2
_NS = 16
_NW = _NC * _NS
_PER_W = _B // _NW
_CHUNK = 128
_NCHUNK = _PER_W // _CHUNK


def _rsqrt_nr(x):
    i = lax.bitcast_convert_type(x, jnp.int32)
    i = 0x5F3759DF - lax.shift_right_arithmetic(i, 1)
    y = lax.bitcast_convert_type(i, jnp.float32)
    for _ in range(2):
        y = y * (1.5 - 0.5 * x * y * y)
    return y


def _row_chunks(ref, i):
    return [ref[i, pl.ds(16 * c, 16)] for c in range(4)]


def _sumsq(chunks):
    v = chunks[0] * chunks[0]
    for c in chunks[1:]:
        v = v + c * c
    return jnp.sum(v)


def _sc_body(ph_h, pr_h, pt_h, nh_h, nr_h, nt_h, ent_h, rel_h, out_h,
             ph_v, pr_v, pt_v, nh_v, nr_v, nt_v,
             hp_r, rp_r, tp_r, hn_r, rn_r, tn_r,
             ost_r, sem):
    wid = lax.axis_index("s") * _NC + lax.axis_index("c")
    base = wid * _PER_W

    pltpu.sync_copy(ph_h.at[pl.ds(base, _PER_W)], ph_v)
    pltpu.sync_copy(pr_h.at[pl.ds(base, _PER_W)], pr_v)
    pltpu.sync_copy(pt_h.at[pl.ds(base, _PER_W)], pt_v)
    pltpu.sync_copy(nh_h.at[pl.ds(base, _PER_W)], nh_v)
    pltpu.sync_copy(nr_h.at[pl.ds(base, _PER_W)], nr_v)
    pltpu.sync_copy(nt_h.at[pl.ds(base, _PER_W)], nt_v)

    acc_total = jnp.float32(0.0)
    for g in range(_NCHUNK):
        sl = pl.ds(g * _CHUNK, _CHUNK)
        cps = [
            pltpu.async_copy(ent_h.at[ph_v.at[sl]], hp_r, sem),
            pltpu.async_copy(rel_h.at[pr_v.at[sl]], rp_r, sem),
            pltpu.async_copy(ent_h.at[pt_v.at[sl]], tp_r, sem),
            pltpu.async_copy(ent_h.at[nh_v.at[sl]], hn_r, sem),
            pltpu.async_copy(rel_h.at[nr_v.at[sl]], rn_r, sem),
            pltpu.async_copy(ent_h.at[nt_v.at[sl]], tn_r, sem),
        ]
        for cp in cps:
            cp.wait()

        def body(i, acc):
            hp = _row_chunks(hp_r, i)
            rp = _row_chunks(rp_r, i)
            tp = _row_chunks(tp_r, i)
            hn = _row_chunks(hn_r, i)
            rn = _row_chunks(rn_r, i)
            tn = _row_chunks(tn_r, i)
            ihp = _rsqrt_nr(_sumsq(hp))
            itp = _rsqrt_nr(_sumsq(tp))
            ihn = _rsqrt_nr(_sumsq(hn))
            itn = _rsqrt_nr(_sumsq(tn))
            dv = None
            for c in range(4):
                p = jnp.abs(hp[c] * ihp + rp[c] - tp[c] * itp)
                n = jnp.abs(hn[c] * ihn + rn[c] - tn[c] * itn)
                d = p - n
                dv = d if dv is None else dv + d
            s = jnp.sum(dv)
            return acc + jnp.maximum(s + _MARGIN, 0.0)

        acc_total = lax.fori_loop(0, _CHUNK, body, acc_total)

    ost_r[...] = lax.broadcast(acc_total, (16,))
    pltpu.sync_copy(ost_r, out_h.at[wid])


@jax.jit
def _sc_call(ph, pr, pt, nh, nr, nt, ent, rel):
    mesh = plsc.VectorSubcoreMesh(core_axis_name="c", subcore_axis_name="s")
    f = pl.kernel(
        _sc_body,
        out_type=jax.ShapeDtypeStruct((_NW, 16), jnp.float32),
        mesh=mesh,
        compiler_params=pltpu.CompilerParams(
            needs_layout_passes=False, use_tc_tiling_on_sc=False),
        scratch_types=[
            pltpu.VMEM((_PER_W,), jnp.int32),
            pltpu.VMEM((_PER_W,), jnp.int32),
            pltpu.VMEM((_PER_W,), jnp.int32),
            pltpu.VMEM((_PER_W,), jnp.int32),
            pltpu.VMEM((_PER_W,), jnp.int32),
            pltpu.VMEM((_PER_W,), jnp.int32),
            pltpu.VMEM((_CHUNK, _D), jnp.float32),
            pltpu.VMEM((_CHUNK, _D), jnp.float32),
            pltpu.VMEM((_CHUNK, _D), jnp.float32),
            pltpu.VMEM((_CHUNK, _D), jnp.float32),
            pltpu.VMEM((_CHUNK, _D), jnp.float32),
            pltpu.VMEM((_CHUNK, _D), jnp.float32),
            pltpu.VMEM((16,), jnp.float32),
            pltpu.SemaphoreType.DMA,
        ],
    )
    return f(ph, pr, pt, nh, nr, nt, ent, rel)


def kernel(pos_batch, neg_batch, entity_emb, relation_emb):
    ph, pr, pt = pos_batch[:, 0], pos_batch[:, 1], pos_batch[:, 2]
    nh, nr, nt = neg_batch[:, 0], neg_batch[:, 1], neg_batch[:, 2]
    partials = _sc_call(ph, pr, pt, nh, nr, nt, entity_emb, relation_emb)
    return jnp.sum(partials[:, 0]) / jnp.float32(_B)
